# Initial kernel scaffold; baseline (speedup 1.0000x reference)
import jax
import jax.numpy as jnp
from jax.experimental import pallas as pl


def kernel(unary, binary, index1, index2):
    raise NotImplementedError("write your pallas kernel here")


# speedup vs baseline: 2.3726x; 2.3726x over previous
import functools

import jax
import jax.numpy as jnp
from jax import lax
from jax.experimental import pallas as pl
from jax.experimental.pallas import tpu as pltpu
from jax.experimental.pallas import tpu_sc as plsc

N_NODES = 10000
N_EDGES = 320000
D_FEAT = 128
D_EDGE = 16
D_OUT = 2 * D_FEAT + D_EDGE

NUM_CORES = 2
NUM_SUBCORES = 16
NW = NUM_CORES * NUM_SUBCORES
B_PER_W = N_EDGES // NW
CHUNK = 200
N_CHUNKS = B_PER_W // CHUNK

_mesh = plsc.VectorSubcoreMesh(core_axis_name="c", subcore_axis_name="s")


@functools.partial(
    pl.kernel,
    mesh=_mesh,
    out_type=jax.ShapeDtypeStruct((N_EDGES, D_OUT), jnp.float32),
    scratch_types=[
        pltpu.VMEM((CHUNK,), jnp.int32),
        pltpu.VMEM((CHUNK,), jnp.int32),
        pltpu.VMEM((CHUNK, D_FEAT), jnp.float32),
        pltpu.VMEM((CHUNK, D_FEAT), jnp.float32),
        pltpu.VMEM((CHUNK, D_EDGE), jnp.float32),
        pltpu.SemaphoreType.DMA,
    ],
)
def _join_sc(unary, binary, index1, index2, out,
             idx1_v, idx2_v, g1_v, g2_v, bin_v, sem):
    wid = lax.axis_index("s") * NUM_CORES + lax.axis_index("c")

    def body(i, carry):
        base = wid * B_PER_W + i * CHUNK
        pltpu.sync_copy(index1.at[pl.ds(base, CHUNK)], idx1_v)
        pltpu.sync_copy(index2.at[pl.ds(base, CHUNK)], idx2_v)
        pltpu.async_copy(unary.at[idx1_v], g1_v, sem).wait()
        pltpu.async_copy(unary.at[idx2_v], g2_v, sem).wait()
        pltpu.sync_copy(binary.at[pl.ds(base, CHUNK)], bin_v)
        pltpu.sync_copy(g1_v, out.at[pl.ds(base, CHUNK), pl.ds(0, D_FEAT)])
        pltpu.sync_copy(g2_v, out.at[pl.ds(base, CHUNK), pl.ds(D_FEAT, D_FEAT)])
        pltpu.sync_copy(bin_v, out.at[pl.ds(base, CHUNK), pl.ds(2 * D_FEAT, D_EDGE)])
        return carry

    lax.fori_loop(0, N_CHUNKS, body, 0)


def kernel(unary, binary, index1, index2):
    return _join_sc(unary, binary, index1, index2)
